# baseline (speedup 1.0000x reference)
import jax
import jax.numpy as jnp
from jax.experimental import pallas as pl
from jax.experimental.pallas import tpu as pltpu

_TB = 8


def _body(img_ref, w_ref, p_ref, o_ref, m_ref):
    @pl.when(pl.program_id(0) == 0)
    def _():
        m_ref[...] = jnp.where(p_ref[...], 0.0, w_ref[...])

    o_ref[...] = img_ref[...] * m_ref[...][None, :, :]


def kernel(image, weight, pruned):
    B, C, H, W = image.shape
    BC = B * C
    img = image.reshape(BC, H, W)
    out = pl.pallas_call(
        _body,
        grid=(BC // _TB,),
        in_specs=[
            pl.BlockSpec((_TB, H, W), lambda i: (i, 0, 0)),
            pl.BlockSpec((H, W), lambda i: (0, 0)),
            pl.BlockSpec((H, W), lambda i: (0, 0)),
        ],
        out_specs=pl.BlockSpec((_TB, H, W), lambda i: (i, 0, 0)),
        out_shape=jax.ShapeDtypeStruct((BC, H, W), image.dtype),
        scratch_shapes=[pltpu.VMEM((H, W), jnp.float32)],
        compiler_params=pltpu.CompilerParams(
            dimension_semantics=("arbitrary",),
        ),
    )(img, weight, pruned)
    return out.reshape(1, B, C, H, W)
